# baseline (speedup 1.0000x reference)
import functools

import jax
import jax.numpy as jnp
from jax import lax
from jax.experimental import pallas as pl
from jax.experimental.pallas import tpu as pltpu

_SCENE_SCALING = 1.0
_BT = 512


def _assemble_body(dens_ref, rgb_ref, grad_ref, sh_ref, gx_ref, gy_ref,
                   gz_ref, neigh_ref, center_ref, out_ref):
    gx = gx_ref[...]
    gy = gy_ref[...]
    gz = gz_ref[...]
    cx = center_ref[0, 0]
    cy = center_ref[0, 1]
    cz = center_ref[0, 2]
    inv = 1.0 / _SCENE_SCALING
    nx = (0.25 * (gx[0] + gx[1] + gx[2] + gx[3]) - cx) * inv
    ny = (0.25 * (gy[0] + gy[1] + gy[2] + gy[3]) - cy) * inv
    nz = (0.25 * (gz[0] + gz[1] + gz[2] + gz[3]) - cz) * inv

    ax, ay, az = gx[1] - gx[0], gy[1] - gy[0], gz[1] - gz[0]
    bx, by, bz = gx[2] - gx[0], gy[2] - gy[0], gz[2] - gz[0]
    cx_, cy_, cz_ = gx[3] - gx[0], gy[3] - gy[0], gz[3] - gz[0]
    det = (ax * (by * cz_ - bz * cy_)
           - ay * (bx * cz_ - bz * cx_)
           + az * (bx * cy_ - by * cx_))
    area = jnp.abs(det) * (1.0 / 6.0)

    dens = dens_ref[...]
    d = jnp.exp(jnp.clip(jnp.log(jnp.clip(dens, 1e-6, None)), None, 15.0))
    grd = jnp.clip(grad_ref[...], -0.99, 0.99)

    out_ref[...] = jnp.concatenate(
        [nx[:, None], ny[:, None], nz[:, None], d[:, None], rgb_ref[...],
         grd, sh_ref[...], area[:, None], neigh_ref[...][:, None]], axis=1)


@jax.jit
def _assemble(density, rgb, gradient, sh, gx, gy, gz, neigh, center):
    T = density.shape[0]
    grid = (T // _BT,)
    return pl.pallas_call(
        _assemble_body,
        grid=grid,
        in_specs=[
            pl.BlockSpec((_BT,), lambda i: (i,)),
            pl.BlockSpec((_BT, 3), lambda i: (i, 0)),
            pl.BlockSpec((_BT, 3), lambda i: (i, 0)),
            pl.BlockSpec((_BT, 24), lambda i: (i, 0)),
            pl.BlockSpec((4, _BT), lambda i: (0, i)),
            pl.BlockSpec((4, _BT), lambda i: (0, i)),
            pl.BlockSpec((4, _BT), lambda i: (0, i)),
            pl.BlockSpec((_BT,), lambda i: (i,)),
            pl.BlockSpec((1, 3), lambda i: (0, 0)),
        ],
        out_specs=pl.BlockSpec((_BT, 36), lambda i: (i, 0)),
        out_shape=jax.ShapeDtypeStruct((T, 36), jnp.float32),
    )(density, rgb, gradient, sh, gx, gy, gz, neigh, center)


def kernel(int_vertices, ext_vertices, density, rgb, gradient, sh, center,
           indices):
    T = indices.shape[0]
    vertices = jnp.concatenate([int_vertices, ext_vertices], axis=0)
    V = vertices.shape[0]
    idx_t = indices.T

    gx = vertices[:, 0][idx_t]
    gy = vertices[:, 1][idx_t]
    gz = vertices[:, 2][idx_t]
    d = jnp.exp(jnp.clip(jnp.log(jnp.clip(density, 1e-6, None)), None, 15.0))
    vd = jnp.zeros((V,), jnp.float32).at[idx_t.reshape(-1)].max(
        jnp.tile(d, 4))
    neigh = vd[idx_t].max(axis=0)

    return _assemble(density, rgb, gradient, sh, gx, gy, gz, neigh, center)
